# baseline (speedup 1.0000x reference)
import functools

import jax
import jax.numpy as jnp
from jax import lax
from jax.experimental import pallas as pl
from jax.experimental.pallas import tpu as pltpu
from jax.experimental.pallas import tpu_sc as plsc

_D = 64
_B = 4096
_K = 20
_J = 5
_NC, _NS = 2, 16
_NW = _NC * _NS
_BPW = _B // _NW
_CB = 4
_NCH = _BPW // _CB
_PAIRS = _CB * _K
_GROUPS = _PAIRS // 16
_NROWS = _CB * _J * _K

_mesh = plsc.VectorSubcoreMesh(core_axis_name="c", subcore_axis_name="s")


def _chunk_buffers():
  return [
      pltpu.VMEM((_PAIRS,), jnp.int32),
      pltpu.VMEM((_PAIRS,), jnp.int32),
      pltpu.VMEM((_CB, _J * _K), jnp.int32),
      pltpu.VMEM((_PAIRS, _D), jnp.bfloat16),
      pltpu.VMEM((_PAIRS, _D), jnp.bfloat16),
      pltpu.VMEM((_NROWS, _D), jnp.bfloat16),
      pltpu.VMEM((_PAIRS,), jnp.float32),
      pltpu.VMEM((_NROWS,), jnp.float32),
      pltpu.SemaphoreType.DMA,
      pltpu.SemaphoreType.DMA,
  ]


@functools.partial(
    pl.kernel,
    out_type=(
        jax.ShapeDtypeStruct((_B * _K,), jnp.float32),
        jax.ShapeDtypeStruct((_B * _J * _K,), jnp.float32),
    ),
    mesh=_mesh,
    compiler_params=pltpu.CompilerParams(
        needs_layout_passes=False, use_tc_tiling_on_sc=False),
    scratch_types=_chunk_buffers() + _chunk_buffers(),
)
def _sgns(c_hbm, o_hbm, neg_hbm, v_hbm, u_hbm, pos_hbm, nout_hbm, *bufs):
  wid = lax.axis_index("s") * _NC + lax.axis_index("c")
  iota = lax.broadcasted_iota(jnp.int32, (16,), 0)
  sets = (bufs[:10], bufs[10:])

  def idx_slices(ch):
    b0 = wid * _BPW + ch * _CB
    p0 = b0 * _K
    return (c_hbm.at[pl.ds(p0, _PAIRS)], o_hbm.at[pl.ds(p0, _PAIRS)],
            neg_hbm.at[pl.ds(b0, _CB)])

  def stage_idx(s, ch):
    cidx, oidx, nidx, sem = s[0], s[1], s[2], s[8]
    csl, osl, nsl = idx_slices(ch)
    pltpu.async_copy(csl, cidx, sem)
    pltpu.async_copy(osl, oidx, sem)
    pltpu.async_copy(nsl, nidx, sem)

  def wait_idx(s, ch):
    cidx, oidx, nidx, sem = s[0], s[1], s[2], s[8]
    csl, osl, nsl = idx_slices(ch)
    pltpu.make_async_copy(csl, cidx, sem).wait()
    pltpu.make_async_copy(osl, oidx, sem).wait()
    pltpu.make_async_copy(nsl, nidx, sem).wait()

  def fire_gathers(s):
    cidx, oidx, nidx, vbuf, ubuf, nbuf, sem = (
        s[0], s[1], s[2], s[3], s[4], s[5], s[9])
    pltpu.async_copy(v_hbm.at[cidx], vbuf, sem)
    pltpu.async_copy(u_hbm.at[oidx], ubuf, sem)
    for i in range(_CB):
      pltpu.async_copy(u_hbm.at[nidx.at[i]],
                       nbuf.at[pl.ds(i * _J * _K, _J * _K)], sem)

  def wait_gathers(s):
    cidx, oidx, nidx, vbuf, ubuf, nbuf, sem = (
        s[0], s[1], s[2], s[3], s[4], s[5], s[9])
    pltpu.make_async_copy(v_hbm.at[cidx], vbuf, sem).wait()
    pltpu.make_async_copy(u_hbm.at[oidx], ubuf, sem).wait()
    for i in range(_CB):
      pltpu.make_async_copy(u_hbm.at[nidx.at[i]],
                            nbuf.at[pl.ds(i * _J * _K, _J * _K)], sem).wait()

  def compute(s, ch):
    vbuf, ubuf, nbuf, posbuf, noutbuf = s[3], s[4], s[5], s[6], s[7]
    b0 = wid * _BPW + ch * _CB
    one = jnp.float32(1.0)

    def group_body(g, carry):
      pg = g * 16
      pvec = pg + iota
      bbv = lax.div(pvec, _K)
      nr0 = bbv * (_J * _K) + (pvec - bbv * _K)

      accp = jnp.zeros((16,), jnp.float32)
      accn = [jnp.zeros((16,), jnp.float32) for _ in range(_J)]
      for i in range(16):
        p = pg + i
        bb = lax.div(p, _K)
        kk = p - bb * _K
        nbase = bb * (_J * _K) + kk
        vv = [plsc.unpack(vbuf[p, pl.ds(32 * t, 32)],
                          format=plsc.PackFormat.INTERLEAVED)
              for t in range(2)]
        uu = [plsc.unpack(ubuf[p, pl.ds(32 * t, 32)],
                          format=plsc.PackFormat.INTERLEAVED)
              for t in range(2)]
        sp = (vv[0][0] * uu[0][0] + vv[0][1] * uu[0][1]
              + vv[1][0] * uu[1][0] + vv[1][1] * uu[1][1])
        accp = jnp.where(iota == i, jnp.sum(sp), accp)
        for j in range(_J):
          nr = nbase + j * _K
          nn = [plsc.unpack(nbuf[nr, pl.ds(32 * t, 32)],
                            format=plsc.PackFormat.INTERLEAVED)
                for t in range(2)]
          sn = (vv[0][0] * nn[0][0] + vv[0][1] * nn[0][1]
                + vv[1][0] * nn[1][0] + vv[1][1] * nn[1][1])
          accn[j] = jnp.where(iota == i, jnp.sum(sn), accn[j])

      posbuf[pl.ds(pg, 16)] = one / (one + jnp.exp(-accp))
      for j in range(_J):
        plsc.store_scatter(noutbuf, [nr0 + j * _K],
                           one / (one + jnp.exp(accn[j])))
      return carry

    lax.fori_loop(0, _GROUPS, group_body, 0)
    pltpu.sync_copy(posbuf, pos_hbm.at[pl.ds(b0 * _K, _PAIRS)])
    pltpu.sync_copy(noutbuf, nout_hbm.at[pl.ds(b0 * (_J * _K), _NROWS)])

  stage_idx(sets[0], 0)
  wait_idx(sets[0], 0)
  fire_gathers(sets[0])
  stage_idx(sets[1], 1)

  def body(i, carry):
    e = 2 * i
    o = e + 1
    wait_idx(sets[1], o)
    fire_gathers(sets[1])

    wait_gathers(sets[0])

    @pl.when(e + 2 < _NCH)
    def _():
      stage_idx(sets[0], e + 2)

    compute(sets[0], e)

    @pl.when(e + 2 < _NCH)
    def _():
      wait_idx(sets[0], e + 2)
      fire_gathers(sets[0])

    wait_gathers(sets[1])

    @pl.when(o + 2 < _NCH)
    def _():
      stage_idx(sets[1], o + 2)

    compute(sets[1], o)
    return carry

  lax.fori_loop(0, _NCH // 2, body, 0)


_V = 100000
_VPW = _V // _NW
_CCH = 625
_NCCH = _VPW // _CCH


@functools.partial(
    pl.kernel,
    out_type=(
        jax.ShapeDtypeStruct((_V, _D), jnp.bfloat16),
        jax.ShapeDtypeStruct((_V, _D), jnp.bfloat16),
    ),
    mesh=_mesh,
    compiler_params=pltpu.CompilerParams(
        needs_layout_passes=False, use_tc_tiling_on_sc=False),
    scratch_types=[
        pltpu.VMEM((_CCH, _D), jnp.float32),
        pltpu.VMEM((_CCH, _D), jnp.float32),
        pltpu.VMEM((_CCH, _D), jnp.bfloat16),
        pltpu.SemaphoreType.DMA,
        pltpu.SemaphoreType.DMA,
    ],
)
def _to_bf16(v_hbm, u_hbm, v16_hbm, u16_hbm, fin_a, fin_b, fout, sem_a,
             sem_b):
  wid = lax.axis_index("s") * _NC + lax.axis_index("c")
  base = wid * _VPW
  fins = (fin_a, fin_b)
  sems = (sem_a, sem_b)

  def do_table(src, dst):
    def slab(c):
      return pl.ds(base + c * _CCH, _CCH)

    pltpu.async_copy(src.at[slab(0)], fins[0], sems[0])
    for c in range(_NCCH):
      fin, sem = fins[c % 2], sems[c % 2]
      if c + 1 < _NCCH:
        pltpu.async_copy(src.at[slab(c + 1)], fins[(c + 1) % 2],
                         sems[(c + 1) % 2])
      pltpu.make_async_copy(src.at[slab(c)], fin, sem).wait()

      def rows(i, carry):
        for r5 in range(5):
          r = 5 * i + r5
          for h in range(2):
            a = fin[r, pl.ds(32 * h, 16)]
            b = fin[r, pl.ds(32 * h + 16, 16)]
            fout[r, pl.ds(32 * h, 32)] = plsc.pack(
                a, b, format=plsc.PackFormat.INTERLEAVED)
        return carry

      lax.fori_loop(0, _CCH // 5, rows, 0)
      pltpu.sync_copy(fout, dst.at[slab(c)])

  do_table(v_hbm, v16_hbm)
  do_table(u_hbm, u16_hbm)


def kernel(c, o, neg, vEmb, uEmb):
  c_f = c.reshape(-1).astype(jnp.int32)
  o_f = o.reshape(-1).astype(jnp.int32)
  neg_f = neg.reshape(_B, _J * _K).astype(jnp.int32)
  v16, u16 = _to_bf16(vEmb, uEmb)
  pos, nout = _sgns(c_f, o_f, neg_f, v16, u16)
  return pos.reshape(_B, _K), nout.reshape(_B, _J, _K)
